# baseline (speedup 1.0000x reference)
import jax
import jax.numpy as jnp
from jax.experimental import pallas as pl
from jax.experimental.pallas import tpu as pltpu

N = 100000
DIM = 128
BLK = 10000


def _body(m_ref, w_ref, x_ref, o_ref):
    x = x_ref[...]
    lhs = m_ref[0]
    rhs = w_ref[...]
    f = jax.lax.dot_general(
        lhs, rhs, (((0,), (0,)), ((), ())),
        preferred_element_type=jnp.float32)
    o_ref[...] = x * f


def kernel(x, node_mask, deletion_weight):
    m = node_mask.astype(jnp.float32).reshape(N // BLK, 1, BLK)
    ones = jnp.ones_like(m)
    m2 = jnp.concatenate([m, ones], axis=1)
    wr = jnp.stack([deletion_weight - 1.0,
                    jnp.ones((DIM,), jnp.float32)], axis=0)
    return pl.pallas_call(
        _body,
        grid=(N // BLK,),
        in_specs=[
            pl.BlockSpec((1, 2, BLK), lambda i: (i, 0, 0)),
            pl.BlockSpec((2, DIM), lambda i: (0, 0)),
            pl.BlockSpec((BLK, DIM), lambda i: (i, 0)),
        ],
        out_specs=pl.BlockSpec((BLK, DIM), lambda i: (i, 0)),
        out_shape=jax.ShapeDtypeStruct((N, DIM), jnp.float32),
        compiler_params=pltpu.CompilerParams(
            dimension_semantics=("parallel",),
            fuse_transposed_lhs_in_matmul=True,
        ),
    )(m2, wr, x)
